# Initial kernel scaffold; baseline (speedup 1.0000x reference)
import functools

import jax
import jax.numpy as jnp
from jax import lax
from jax.experimental import pallas as pl
from jax.experimental.pallas import tpu as pltpu
from jax.experimental.pallas import tpu_sc as plsc

N_NODES = 10000
D = 128
N_GRAPHS = 128

NROWS = 10240
NB = NROWS // 128
BM = 128

N_TILES = 32
K = 128
CH = 79
NE_PAD = N_TILES * CH * K
RPT = NROWS // 16


def _sc_segment_sum(h_pad, src_r, dst_r, zeros):
    mesh = plsc.VectorSubcoreMesh(core_axis_name="c", subcore_axis_name="s")

    @functools.partial(
        pl.kernel,
        mesh=mesh,
        out_type=jax.ShapeDtypeStruct((2, NROWS, D), jnp.float32),
        scratch_types=[
            pltpu.VMEM_SHARED((NROWS, D), jnp.float32),
            pltpu.VMEM((CH, K), jnp.int32),
            pltpu.VMEM((CH, K), jnp.int32),
            pltpu.VMEM((K, D), jnp.float32),
            pltpu.SemaphoreType.DMA,
        ],
    )
    def k(h_hbm, src_hbm, dst_hbm, z_hbm, out_hbm, agg_sh, src_v, dst_v, rows_v, sem):
        c = lax.axis_index("c")
        s = lax.axis_index("s")
        t = c * 16 + s

        pltpu.sync_copy(z_hbm.at[pl.ds(s * RPT, RPT)], agg_sh.at[pl.ds(s * RPT, RPT)])
        pltpu.sync_copy(src_hbm.at[t], src_v)
        pltpu.sync_copy(dst_hbm.at[t], dst_v)
        plsc.subcore_barrier()

        def chunk(i, carry):
            pltpu.async_copy(h_hbm.at[src_v.at[i]], rows_v, sem).wait()
            pltpu.sync_copy(rows_v, agg_sh.at[dst_v.at[i]], add=True)
            return carry

        lax.fori_loop(0, CH, chunk, 0)
        plsc.subcore_barrier()
        pltpu.sync_copy(agg_sh.at[pl.ds(s * RPT, RPT)],
                        out_hbm.at[c, pl.ds(s * RPT, RPT)])

    return k(h_pad, src_r, dst_r, zeros)


def _mlp_body(h_ref, agg_ref, w1_ref, b1_ref, w2_ref, b2_ref,
              u_ref, sums_ref, sumsq_ref):
    i = pl.program_id(0)
    z = h_ref[...] + agg_ref[0] + agg_ref[1]
    t1 = jnp.maximum(jnp.dot(z, w1_ref[...], preferred_element_type=jnp.float32)
                     + b1_ref[...], 0.0)
    u = jnp.maximum(jnp.dot(t1, w2_ref[...], preferred_element_type=jnp.float32)
                    + b2_ref[...], 0.0)
    rows = i * BM + lax.broadcasted_iota(jnp.int32, (BM, 1), 0)
    u = jnp.where(rows < N_NODES, u, 0.0)
    u_ref[...] = u
    sums_ref[...] = jnp.sum(u, axis=0, keepdims=True)
    sumsq_ref[...] = jnp.sum(u * u, axis=0, keepdims=True)


def _mlp(h, agg, W1, b1, W2, b2):
    return pl.pallas_call(
        _mlp_body,
        grid=(NB,),
        in_specs=[
            pl.BlockSpec((BM, D), lambda i: (i, 0)),
            pl.BlockSpec((2, BM, D), lambda i: (0, i, 0)),
            pl.BlockSpec((D, D), lambda i: (0, 0)),
            pl.BlockSpec((1, D), lambda i: (0, 0)),
            pl.BlockSpec((D, D), lambda i: (0, 0)),
            pl.BlockSpec((1, D), lambda i: (0, 0)),
        ],
        out_specs=[
            pl.BlockSpec((BM, D), lambda i: (i, 0)),
            pl.BlockSpec((1, D), lambda i: (i, 0)),
            pl.BlockSpec((1, D), lambda i: (i, 0)),
        ],
        out_shape=[
            jax.ShapeDtypeStruct((NROWS, D), jnp.float32),
            jax.ShapeDtypeStruct((NB, D), jnp.float32),
            jax.ShapeDtypeStruct((NB, D), jnp.float32),
        ],
    )(h, agg, W1, b1, W2, b2)


def _bn_body(u_ref, sums_ref, sumsq_ref, g_ref, b_ref, out_ref):
    i = pl.program_id(0)
    m = jnp.sum(sums_ref[...], axis=0, keepdims=True) / N_NODES
    v = jnp.sum(sumsq_ref[...], axis=0, keepdims=True) / N_NODES - m * m
    scale = g_ref[...] * lax.rsqrt(v + 1e-5)
    shift = b_ref[...] - m * scale
    out = u_ref[...] * scale + shift
    rows = i * BM + lax.broadcasted_iota(jnp.int32, (BM, 1), 0)
    out_ref[...] = jnp.where(rows < N_NODES, out, 0.0)


def _bn(u, sums, sumsq, g, b):
    return pl.pallas_call(
        _bn_body,
        grid=(NB,),
        in_specs=[
            pl.BlockSpec((BM, D), lambda i: (i, 0)),
            pl.BlockSpec((NB, D), lambda i: (0, 0)),
            pl.BlockSpec((NB, D), lambda i: (0, 0)),
            pl.BlockSpec((1, D), lambda i: (0, 0)),
            pl.BlockSpec((1, D), lambda i: (0, 0)),
        ],
        out_specs=pl.BlockSpec((BM, D), lambda i: (i, 0)),
        out_shape=jax.ShapeDtypeStruct((NROWS, D), jnp.float32),
    )(u, sums, sumsq, g, b)


def _pool_body(h_ref, batch_ref, wfc_ref, bfc_ref, out_ref, acc_ref, cnt_ref):
    i = pl.program_id(0)

    @pl.when(i == 0)
    def _():
        acc_ref[...] = jnp.zeros_like(acc_ref)
        cnt_ref[...] = jnp.zeros_like(cnt_ref)

    bids = batch_ref[0]
    g_col = lax.broadcasted_iota(jnp.int32, (N_GRAPHS, 1), 0)
    mt = (g_col == bids).astype(jnp.float32)
    acc_ref[...] += jnp.dot(mt, h_ref[...], preferred_element_type=jnp.float32)
    cnt_ref[...] += jnp.sum(mt, axis=1, keepdims=True)

    @pl.when(i == NB - 1)
    def _():
        pooled = acc_ref[...] / jnp.maximum(cnt_ref[...], 1.0)
        out = jnp.dot(pooled, wfc_ref[...], preferred_element_type=jnp.float32)
        out_ref[...] = jnp.tanh(out + bfc_ref[...])


def _pool(h, batch_r, Wfc, bfc):
    return pl.pallas_call(
        _pool_body,
        grid=(NB,),
        in_specs=[
            pl.BlockSpec((BM, D), lambda i: (i, 0)),
            pl.BlockSpec((1, 1, K), lambda i: (i, 0, 0)),
            pl.BlockSpec((D, D), lambda i: (0, 0)),
            pl.BlockSpec((1, D), lambda i: (0, 0)),
        ],
        out_specs=pl.BlockSpec((N_GRAPHS, D), lambda i: (0, 0)),
        out_shape=jax.ShapeDtypeStruct((N_GRAPHS, D), jnp.float32),
        scratch_shapes=[
            pltpu.VMEM((N_GRAPHS, D), jnp.float32),
            pltpu.VMEM((N_GRAPHS, 1), jnp.float32),
        ],
    )(h, batch_r, Wfc, bfc)


def kernel(x, edge_index, batch, params):
    src = edge_index[0]
    dst = edge_index[1]
    ne = src.shape[0]
    pad = NE_PAD - ne
    src_r = jnp.concatenate(
        [src, jnp.full((pad,), N_NODES, jnp.int32)]).reshape(N_TILES, CH, K)
    dst_r = jnp.concatenate(
        [dst, jnp.zeros((pad,), jnp.int32)]).reshape(N_TILES, CH, K)
    zeros = jnp.zeros((NROWS, D), jnp.float32)
    h = zeros.at[:N_NODES].set(x)
    batch_r = jnp.concatenate(
        [batch, jnp.full((NROWS - N_NODES,), N_GRAPHS, jnp.int32)]
    ).reshape(NB, 1, K)

    for l in range(5):
        W1, b1, W2, b2 = params["convs"][l]
        g, bb = params["bns"][l]
        agg = _sc_segment_sum(h, src_r, dst_r, zeros)
        u, sums, sumsq = _mlp(h, agg, W1, b1.reshape(1, D), W2, b2.reshape(1, D))
        h = _bn(u, sums, sumsq, g.reshape(1, D), bb.reshape(1, D))

    Wfc, bfc = params["fc"]
    return _pool(h, batch_r, Wfc, bfc.reshape(1, D))


# speedup vs baseline: 2.2759x; 2.2759x over previous
import functools

import jax
import jax.numpy as jnp
from jax import lax
from jax.experimental import pallas as pl
from jax.experimental.pallas import tpu as pltpu
from jax.experimental.pallas import tpu_sc as plsc

N_NODES = 10000
D = 128
N_GRAPHS = 128
E = 320000

NROWS = 10240
NB = NROWS // 128
BM = 128

K = 128
E_PAD = 327680
RPT = NROWS // 16
PPT = E_PAD // 32

HALF = E // 2
W_FULL = 42 * 240
W_SMALL = 41 * 240

P_FULL = 3 * 240
P_SMALL = 2 * 240


def _chunk_bounds(s, half, w_full, w_small, n_full):
    lo = w_full * jnp.minimum(s, n_full) + w_small * jnp.maximum(s - n_full, 0)
    s1 = s + 1
    hi = w_full * jnp.minimum(s1, n_full) + w_small * jnp.maximum(s1 - n_full, 0)
    return lo, jnp.minimum(hi, half)


def _sc_prep(src, dst, perm):
    mesh = plsc.VectorSubcoreMesh(core_axis_name="c", subcore_axis_name="s")

    @functools.partial(
        pl.kernel,
        mesh=mesh,
        out_type=[
            jax.ShapeDtypeStruct((E_PAD,), jnp.int32),
            jax.ShapeDtypeStruct((E_PAD,), jnp.int32),
        ],
        scratch_types=[
            pltpu.VMEM((K,), jnp.int32),
            pltpu.VMEM((K,), jnp.int32),
            pltpu.SemaphoreType.DMA,
        ],
    )
    def k(src_hbm, dst_hbm, perm_hbm, ssrc_hbm, sdst_hbm, pidx, vals, sem):
        c = lax.axis_index("c")
        s = lax.axis_index("s")
        t = c * 16 + s

        def chunk(i, carry):
            base = t * PPT + i * K
            pltpu.sync_copy(perm_hbm.at[pl.ds(base, K)], pidx)
            pltpu.async_copy(src_hbm.at[pidx], vals, sem).wait()
            pltpu.sync_copy(vals, ssrc_hbm.at[pl.ds(base, K)])
            pltpu.async_copy(dst_hbm.at[pidx], vals, sem).wait()
            pltpu.sync_copy(vals, sdst_hbm.at[pl.ds(base, K)])
            return carry

        lax.fori_loop(0, PPT // K, chunk, 0)

    return k(src, dst, perm)


def _seq_segsum_chunk(lo, hi, rows_fetch, dst_hbm, agg_sh, dstv, stage, fidx):
    nblk = (hi - lo + K - 1) // K
    lanes = lax.broadcasted_iota(jnp.int32, (16,), 0)

    def flush(cur, acc):
        b = cur * D
        for jj in range(8):
            fidx[pl.ds(16 * jj, 16)] = b + (16 * jj) + lanes
            stage[pl.ds(16 * jj, 16)] = acc[jj]
        pltpu.sync_copy(stage, agg_sh.at[fidx], add=True)

    def sub(bi, carry):
        base = lo + bi * K
        rows = rows_fetch(base)
        pltpu.sync_copy(dst_hbm.at[pl.ds(base, K)], dstv)

        def grp(gi, car):
            gbase = gi * 16
            dvec = dstv[pl.ds(gbase, 16)]
            cur = car[0]
            acc = list(car[1:])
            for j in range(16):
                u = gbase + j
                valid = (base + u) < hi
                d = dvec[j]
                is_new = jnp.logical_and(d != cur, valid)
                fl = jnp.logical_and(is_new, cur >= 0)

                @pl.when(fl)
                def _(cur=cur, acc=tuple(acc)):
                    flush(cur, acc)

                newacc = []
                for jj in range(8):
                    r = rows[u, pl.ds(16 * jj, 16)]
                    a = jnp.where(is_new, r,
                                  jnp.where(valid, acc[jj] + r, acc[jj]))
                    newacc.append(a)
                acc = newacc
                cur = jnp.where(valid, d, cur)
            return tuple([cur] + acc)

        return lax.fori_loop(0, K // 16, grp, carry)

    init = tuple([jnp.int32(-1)] + [jnp.zeros((16,), jnp.float32)] * 8)
    final = lax.fori_loop(0, nblk, sub, init)

    @pl.when(final[0] >= 0)
    def _():
        flush(final[0], list(final[1:]))


def _sc_agg(h_pad, ssrc, sdst, zeros):
    mesh = plsc.VectorSubcoreMesh(core_axis_name="c", subcore_axis_name="s")

    @functools.partial(
        pl.kernel,
        mesh=mesh,
        out_type=jax.ShapeDtypeStruct((2, NROWS * D), jnp.float32),
        scratch_types=[
            pltpu.VMEM_SHARED((NROWS * D,), jnp.float32),
            pltpu.VMEM((K,), jnp.int32),
            pltpu.VMEM((K,), jnp.int32),
            pltpu.VMEM((K, D), jnp.float32),
            pltpu.VMEM((D,), jnp.float32),
            pltpu.VMEM((D,), jnp.int32),
            pltpu.SemaphoreType.DMA,
        ],
    )
    def k(h_hbm, src_hbm, dst_hbm, z_hbm, out_hbm,
          agg_sh, srcv, dstv, rows, stage, fidx, sem):
        c = lax.axis_index("c")
        s = lax.axis_index("s")
        pltpu.sync_copy(z_hbm.at[pl.ds(s * RPT * D, RPT * D)],
                        agg_sh.at[pl.ds(s * RPT * D, RPT * D)])
        plsc.subcore_barrier()

        lo, hi = _chunk_bounds(s, HALF, W_FULL, W_SMALL, 11)
        lo = lo + c * HALF
        hi = hi + c * HALF

        def rows_fetch(base):
            pltpu.sync_copy(src_hbm.at[pl.ds(base, K)], srcv)
            pltpu.async_copy(h_hbm.at[srcv], rows, sem).wait()
            return rows

        _seq_segsum_chunk(lo, hi, rows_fetch, dst_hbm, agg_sh, dstv, stage, fidx)

        plsc.subcore_barrier()
        pltpu.sync_copy(agg_sh.at[pl.ds(s * RPT * D, RPT * D)],
                        out_hbm.at[c, pl.ds(s * RPT * D, RPT * D)])

    return k(h_pad, ssrc, sdst, zeros)


def _sc_pool(h_pad, batch_pad, zeros):
    mesh = plsc.VectorSubcoreMesh(core_axis_name="c", subcore_axis_name="s")

    @functools.partial(
        pl.kernel,
        mesh=mesh,
        out_type=jax.ShapeDtypeStruct((N_GRAPHS * D,), jnp.float32),
        scratch_types=[
            pltpu.VMEM_SHARED((N_GRAPHS * D,), jnp.float32),
            pltpu.VMEM((K,), jnp.int32),
            pltpu.VMEM((K, D), jnp.float32),
            pltpu.VMEM((D,), jnp.float32),
            pltpu.VMEM((D,), jnp.int32),
            pltpu.SemaphoreType.DMA,
        ],
    )
    def k(h_hbm, b_hbm, z_hbm, out_hbm, agg_sh, dstv, rows, stage, fidx, sem):
        c = lax.axis_index("c")
        s = lax.axis_index("s")

        @pl.when(c == 0)
        def _():
            rpt = (N_GRAPHS // 16) * D
            pltpu.sync_copy(z_hbm.at[pl.ds(s * rpt, rpt)],
                            agg_sh.at[pl.ds(s * rpt, rpt)])
            plsc.subcore_barrier()

            lo, hi = _chunk_bounds(s, N_NODES, P_FULL, P_SMALL, 10)

            def rows_fetch(base):
                pltpu.sync_copy(h_hbm.at[pl.ds(base, K)], rows)
                return rows

            _seq_segsum_chunk(lo, hi, rows_fetch, b_hbm, agg_sh, dstv,
                              stage, fidx)

            plsc.subcore_barrier()
            pltpu.sync_copy(agg_sh.at[pl.ds(s * rpt, rpt)],
                            out_hbm.at[pl.ds(s * rpt, rpt)])

    return k(h_pad, batch_pad, zeros)


def _tree8(acc):
    b = acc[0:4] + acc[4:8]
    c = b[0:2] + b[2:4]
    return c[0:1] + c[1:2]


def _mlp_body(h_ref, agg_ref, w1_ref, b1_ref, w2_ref, b2_ref,
              u_ref, sums_ref, acc_ref):
    i = pl.program_id(0)

    @pl.when(i == 0)
    def _():
        acc_ref[...] = jnp.zeros_like(acc_ref)

    z = h_ref[...] + (agg_ref[0] + agg_ref[1])
    t1 = jnp.maximum(jnp.dot(z, w1_ref[...],
                             preferred_element_type=jnp.float32) + b1_ref[...],
                     0.0)
    u = jnp.maximum(jnp.dot(t1, w2_ref[...],
                            preferred_element_type=jnp.float32) + b2_ref[...],
                    0.0)
    rows = i * BM + lax.broadcasted_iota(jnp.int32, (BM, 1), 0)
    u = jnp.where(rows < N_NODES, u, 0.0)
    u_ref[...] = u

    acc = acc_ref[...]
    for j in range(16):
        acc = acc + u[8 * j:8 * j + 8, :]
    acc_ref[...] = acc

    @pl.when(i == NB - 1)
    def _():
        sums_ref[...] = acc_ref[...]


def _mlp(h, agg, W1, b1, W2, b2):
    return pl.pallas_call(
        _mlp_body,
        grid=(NB,),
        in_specs=[
            pl.BlockSpec((BM, D), lambda i: (i, 0)),
            pl.BlockSpec((2, BM, D), lambda i: (0, i, 0)),
            pl.BlockSpec((D, D), lambda i: (0, 0)),
            pl.BlockSpec((1, D), lambda i: (0, 0)),
            pl.BlockSpec((D, D), lambda i: (0, 0)),
            pl.BlockSpec((1, D), lambda i: (0, 0)),
        ],
        out_specs=[
            pl.BlockSpec((BM, D), lambda i: (i, 0)),
            pl.BlockSpec((8, D), lambda i: (0, 0)),
        ],
        out_shape=[
            jax.ShapeDtypeStruct((NROWS, D), jnp.float32),
            jax.ShapeDtypeStruct((8, D), jnp.float32),
        ],
        scratch_shapes=[pltpu.VMEM((8, D), jnp.float32)],
    )(h, agg, W1, b1, W2, b2)


def _var_body(u_ref, sums_ref, v0_ref, v1_ref, a0_ref, a1_ref):
    i = pl.program_id(0)

    @pl.when(i == 0)
    def _():
        a0_ref[...] = jnp.zeros_like(a0_ref)
        a1_ref[...] = jnp.zeros_like(a1_ref)

    m = _tree8(sums_ref[...]) * jnp.float32(1e-4)
    rows = i * BM + lax.broadcasted_iota(jnp.int32, (BM, 1), 0)
    dev = u_ref[...] - m
    sq = jnp.where(rows < N_NODES, dev * dev, 0.0)

    @pl.when(i < 39)
    def _():
        acc = a0_ref[...]
        for j in range(16):
            acc = acc + sq[8 * j:8 * j + 8, :]
        a0_ref[...] = acc

    @pl.when(i == 39)
    def _():
        a0_ref[...] = a0_ref[...] + sq[0:8, :]
        acc = a1_ref[...]
        for j in range(1, 16):
            acc = acc + sq[8 * j:8 * j + 8, :]
        a1_ref[...] = acc

    @pl.when(i > 39)
    def _():
        acc = a1_ref[...]
        for j in range(16):
            acc = acc + sq[8 * j:8 * j + 8, :]
        a1_ref[...] = acc

    @pl.when(i == NB - 1)
    def _():
        v0_ref[...] = a0_ref[...]
        v1_ref[...] = a1_ref[...]


def _var(u, sums):
    return pl.pallas_call(
        _var_body,
        grid=(NB,),
        in_specs=[
            pl.BlockSpec((BM, D), lambda i: (i, 0)),
            pl.BlockSpec((8, D), lambda i: (0, 0)),
        ],
        out_specs=[
            pl.BlockSpec((8, D), lambda i: (0, 0)),
            pl.BlockSpec((8, D), lambda i: (0, 0)),
        ],
        out_shape=[
            jax.ShapeDtypeStruct((8, D), jnp.float32),
            jax.ShapeDtypeStruct((8, D), jnp.float32),
        ],
        scratch_shapes=[
            pltpu.VMEM((8, D), jnp.float32),
            pltpu.VMEM((8, D), jnp.float32),
        ],
    )(u, sums)


def _norm_body(u_ref, sums_ref, v0_ref, v1_ref, g_ref, b_ref, out_ref):
    i = pl.program_id(0)
    m = _tree8(sums_ref[...]) * jnp.float32(1e-4)
    vs = _tree8(v0_ref[...]) + _tree8(v1_ref[...])
    den = jnp.sqrt(vs * jnp.float32(1e-4) + jnp.float32(1e-5))
    out = (u_ref[...] - m) / den * g_ref[...] + b_ref[...]
    rows = i * BM + lax.broadcasted_iota(jnp.int32, (BM, 1), 0)
    out_ref[...] = jnp.where(rows < N_NODES, out, 0.0)


def _norm(u, sums, v0, v1, g, b):
    return pl.pallas_call(
        _norm_body,
        grid=(NB,),
        in_specs=[
            pl.BlockSpec((BM, D), lambda i: (i, 0)),
            pl.BlockSpec((8, D), lambda i: (0, 0)),
            pl.BlockSpec((8, D), lambda i: (0, 0)),
            pl.BlockSpec((8, D), lambda i: (0, 0)),
            pl.BlockSpec((1, D), lambda i: (0, 0)),
            pl.BlockSpec((1, D), lambda i: (0, 0)),
        ],
        out_specs=pl.BlockSpec((BM, D), lambda i: (i, 0)),
        out_shape=jax.ShapeDtypeStruct((NROWS, D), jnp.float32),
    )(u, sums, v0, v1, g, b)


def _fc_body(ps_ref, batch_ref, wfc_ref, bfc_ref, out_ref, cnt_ref):
    i = pl.program_id(0)

    @pl.when(i == 0)
    def _():
        cnt_ref[...] = jnp.zeros_like(cnt_ref)

    bids = batch_ref[0]
    g_col = lax.broadcasted_iota(jnp.int32, (N_GRAPHS, 1), 0)
    mt = (g_col == bids).astype(jnp.float32)
    cnt_ref[...] += jnp.sum(mt, axis=1, keepdims=True)

    @pl.when(i == NB - 1)
    def _():
        pooled = ps_ref[...] / jnp.maximum(cnt_ref[...], 1.0)
        out = jnp.dot(pooled, wfc_ref[...],
                      preferred_element_type=jnp.float32)
        out_ref[...] = jnp.tanh(out + bfc_ref[...])


def _fc(pooled_sum, batch_r, Wfc, bfc):
    return pl.pallas_call(
        _fc_body,
        grid=(NB,),
        in_specs=[
            pl.BlockSpec((N_GRAPHS, D), lambda i: (0, 0)),
            pl.BlockSpec((1, 1, BM), lambda i: (i, 0, 0)),
            pl.BlockSpec((D, D), lambda i: (0, 0)),
            pl.BlockSpec((1, D), lambda i: (0, 0)),
        ],
        out_specs=pl.BlockSpec((N_GRAPHS, D), lambda i: (0, 0)),
        out_shape=jax.ShapeDtypeStruct((N_GRAPHS, D), jnp.float32),
        scratch_shapes=[pltpu.VMEM((N_GRAPHS, 1), jnp.float32)],
    )(pooled_sum, batch_r, Wfc, bfc)


def kernel(x, edge_index, batch, params):
    src = edge_index[0]
    dst = edge_index[1]
    perm = jnp.argsort(dst, stable=True).astype(jnp.int32)
    perm_pad = jnp.concatenate(
        [perm, jnp.zeros((E_PAD - E,), jnp.int32)])
    ssrc, sdst = _sc_prep(src, dst, perm_pad)

    zflat = jnp.zeros((NROWS * D,), jnp.float32)
    h = jnp.zeros((NROWS, D), jnp.float32).at[:N_NODES].set(x)
    batch_pad = jnp.concatenate(
        [batch, jnp.full((NROWS - N_NODES,), N_GRAPHS, jnp.int32)])
    batch_r = batch_pad.reshape(NB, 1, BM)

    for l in range(5):
        W1, b1, W2, b2 = params["convs"][l]
        g, bb = params["bns"][l]
        agg = _sc_agg(h, ssrc, sdst, zflat).reshape(2, NROWS, D)
        u, sums = _mlp(h, agg, W1, b1.reshape(1, D), W2, b2.reshape(1, D))
        v0, v1 = _var(u, sums)
        h = _norm(u, sums, v0, v1, g.reshape(1, D), bb.reshape(1, D))

    pooled_sum = _sc_pool(h, batch_pad, zflat).reshape(N_GRAPHS, D)
    Wfc, bfc = params["fc"]
    return _fc(pooled_sum, batch_r, Wfc, bfc.reshape(1, D))


# speedup vs baseline: 2.3848x; 1.0479x over previous
import functools

import jax
import jax.numpy as jnp
from jax import lax
from jax.experimental import pallas as pl
from jax.experimental.pallas import tpu as pltpu
from jax.experimental.pallas import tpu_sc as plsc

N_NODES = 10000
D = 128
N_GRAPHS = 128
E = 320000

NROWS = 10240
NB = NROWS // 128
BM = 128

K = 128
E_PAD = 327680
RPT = NROWS // 16
PPT = E_PAD // 32

HALF = E // 2
W_FULL = 42 * 240
W_SMALL = 41 * 240

P_FULL = 3 * 240
P_SMALL = 2 * 240


def _chunk_bounds(s, half, w_full, w_small, n_full):
    lo = w_full * jnp.minimum(s, n_full) + w_small * jnp.maximum(s - n_full, 0)
    s1 = s + 1
    hi = w_full * jnp.minimum(s1, n_full) + w_small * jnp.maximum(s1 - n_full, 0)
    return lo, jnp.minimum(hi, half)


def _sc_prep(src, dst, perm):
    mesh = plsc.VectorSubcoreMesh(core_axis_name="c", subcore_axis_name="s")

    @functools.partial(
        pl.kernel,
        mesh=mesh,
        out_type=[
            jax.ShapeDtypeStruct((E_PAD,), jnp.int32),
            jax.ShapeDtypeStruct((E_PAD,), jnp.int32),
        ],
        scratch_types=[
            pltpu.VMEM((K,), jnp.int32),
            pltpu.VMEM((K,), jnp.int32),
            pltpu.SemaphoreType.DMA,
        ],
    )
    def k(src_hbm, dst_hbm, perm_hbm, ssrc_hbm, sdst_hbm, pidx, vals, sem):
        c = lax.axis_index("c")
        s = lax.axis_index("s")
        t = c * 16 + s

        def chunk(i, carry):
            base = t * PPT + i * K
            pltpu.sync_copy(perm_hbm.at[pl.ds(base, K)], pidx)
            pltpu.async_copy(src_hbm.at[pidx], vals, sem).wait()
            pltpu.sync_copy(vals, ssrc_hbm.at[pl.ds(base, K)])
            pltpu.async_copy(dst_hbm.at[pidx], vals, sem).wait()
            pltpu.sync_copy(vals, sdst_hbm.at[pl.ds(base, K)])
            return carry

        lax.fori_loop(0, PPT // K, chunk, 0)

    return k(src, dst, perm)


def _seq_segsum_chunk(lo, hi, fetch_start, fetch_wait, rows2, dst_hbm,
                      agg_sh, dstv, stage, fidx):
    nblk = (hi - lo + K - 1) // K
    lanes = lax.broadcasted_iota(jnp.int32, (16,), 0)

    def flush(cur, acc):
        b = cur * D
        for jj in range(8):
            fidx[pl.ds(16 * jj, 16)] = b + (16 * jj) + lanes
            stage[pl.ds(16 * jj, 16)] = acc[jj]
        pltpu.sync_copy(stage, agg_sh.at[fidx], add=True)

    def make_sub(full):
        def sub(bi, carry):
            base = lo + bi * K
            boff = (bi % 2) * K
            fetch_wait(boff)

            @pl.when(bi + 1 < nblk)
            def _():
                fetch_start(bi + 1, (bi + 1) % 2 * K)

            pltpu.sync_copy(dst_hbm.at[pl.ds(base, K)], dstv)

            def grp(gi, car):
                gbase = gi * 16
                dvec = dstv[pl.ds(gbase, 16)]
                cur = car[0]
                acc = list(car[1:])
                for j in range(16):
                    u = boff + gbase + j
                    d = dvec[j]
                    if full:
                        is_new = d != cur
                    else:
                        valid = (base + gbase + j) < hi
                        is_new = jnp.logical_and(d != cur, valid)
                    fl = jnp.logical_and(is_new, cur >= 0)

                    @pl.when(fl)
                    def _(cur=cur, acc=tuple(acc)):
                        flush(cur, acc)

                    newacc = []
                    for jj in range(8):
                        r = rows2[u, pl.ds(16 * jj, 16)]
                        if full:
                            a = jnp.where(is_new, r, acc[jj] + r)
                        else:
                            a = jnp.where(is_new, r,
                                          jnp.where(valid, acc[jj] + r,
                                                    acc[jj]))
                        newacc.append(a)
                    acc = newacc
                    cur = d if full else jnp.where(valid, d, cur)
                return tuple([cur] + acc)

            return lax.fori_loop(0, K // 16, grp, carry)

        return sub

    fetch_start(0, 0)
    init = tuple([jnp.int32(-1)] + [jnp.zeros((16,), jnp.float32)] * 8)
    final = lax.fori_loop(0, nblk - 1, make_sub(True), init)
    final = make_sub(False)(nblk - 1, final)

    @pl.when(final[0] >= 0)
    def _():
        flush(final[0], list(final[1:]))


def _sc_agg(h_pad, ssrc, sdst, zeros):
    mesh = plsc.VectorSubcoreMesh(core_axis_name="c", subcore_axis_name="s")

    @functools.partial(
        pl.kernel,
        mesh=mesh,
        out_type=jax.ShapeDtypeStruct((2, NROWS * D), jnp.float32),
        scratch_types=[
            pltpu.VMEM_SHARED((NROWS * D,), jnp.float32),
            pltpu.VMEM((2 * K,), jnp.int32),
            pltpu.VMEM((K,), jnp.int32),
            pltpu.VMEM((2 * K, D), jnp.float32),
            pltpu.VMEM((D,), jnp.float32),
            pltpu.VMEM((D,), jnp.int32),
            pltpu.SemaphoreType.DMA,
        ],
    )
    def k(h_hbm, src_hbm, dst_hbm, z_hbm, out_hbm,
          agg_sh, srcv, dstv, rows, stage, fidx, sem):
        c = lax.axis_index("c")
        s = lax.axis_index("s")
        pltpu.sync_copy(z_hbm.at[pl.ds(s * RPT * D, RPT * D)],
                        agg_sh.at[pl.ds(s * RPT * D, RPT * D)])
        plsc.subcore_barrier()

        lo, hi = _chunk_bounds(s, HALF, W_FULL, W_SMALL, 11)
        lo = lo + c * HALF
        hi = hi + c * HALF

        def fetch_start(bi, boff):
            pltpu.sync_copy(src_hbm.at[pl.ds(lo + bi * K, K)],
                            srcv.at[pl.ds(boff, K)])
            pltpu.async_copy(h_hbm.at[srcv.at[pl.ds(boff, K)]],
                             rows.at[pl.ds(boff, K)], sem)

        def fetch_wait(boff):
            pltpu.make_async_copy(h_hbm.at[pl.ds(0, K)],
                                  rows.at[pl.ds(boff, K)], sem).wait()

        _seq_segsum_chunk(lo, hi, fetch_start, fetch_wait, rows, dst_hbm,
                          agg_sh, dstv, stage, fidx)

        plsc.subcore_barrier()
        pltpu.sync_copy(agg_sh.at[pl.ds(s * RPT * D, RPT * D)],
                        out_hbm.at[c, pl.ds(s * RPT * D, RPT * D)])

    return k(h_pad, ssrc, sdst, zeros)


def _sc_pool(h_pad, batch_pad, zeros):
    mesh = plsc.VectorSubcoreMesh(core_axis_name="c", subcore_axis_name="s")

    @functools.partial(
        pl.kernel,
        mesh=mesh,
        out_type=jax.ShapeDtypeStruct((N_GRAPHS * D,), jnp.float32),
        scratch_types=[
            pltpu.VMEM_SHARED((N_GRAPHS * D,), jnp.float32),
            pltpu.VMEM((K,), jnp.int32),
            pltpu.VMEM((2 * K, D), jnp.float32),
            pltpu.VMEM((D,), jnp.float32),
            pltpu.VMEM((D,), jnp.int32),
            pltpu.SemaphoreType.DMA,
        ],
    )
    def k(h_hbm, b_hbm, z_hbm, out_hbm, agg_sh, dstv, rows, stage, fidx, sem):
        c = lax.axis_index("c")
        s = lax.axis_index("s")

        @pl.when(c == 0)
        def _():
            rpt = (N_GRAPHS // 16) * D
            pltpu.sync_copy(z_hbm.at[pl.ds(s * rpt, rpt)],
                            agg_sh.at[pl.ds(s * rpt, rpt)])
            plsc.subcore_barrier()

            lo, hi = _chunk_bounds(s, N_NODES, P_FULL, P_SMALL, 10)

            def fetch_start(bi, boff):
                pltpu.async_copy(h_hbm.at[pl.ds(lo + bi * K, K)],
                                 rows.at[pl.ds(boff, K)], sem)

            def fetch_wait(boff):
                pltpu.make_async_copy(h_hbm.at[pl.ds(0, K)],
                                      rows.at[pl.ds(boff, K)], sem).wait()

            _seq_segsum_chunk(lo, hi, fetch_start, fetch_wait, rows, b_hbm,
                              agg_sh, dstv, stage, fidx)

            plsc.subcore_barrier()
            pltpu.sync_copy(agg_sh.at[pl.ds(s * rpt, rpt)],
                            out_hbm.at[pl.ds(s * rpt, rpt)])

    return k(h_pad, batch_pad, zeros)


def _tree8(acc):
    b = acc[0:4] + acc[4:8]
    c = b[0:2] + b[2:4]
    return c[0:1] + c[1:2]


def _mlp_body(h_ref, agg_ref, w1_ref, b1_ref, w2_ref, b2_ref,
              u_ref, sums_ref, acc_ref):
    i = pl.program_id(0)

    @pl.when(i == 0)
    def _():
        acc_ref[...] = jnp.zeros_like(acc_ref)

    z = h_ref[...] + (agg_ref[0] + agg_ref[1])
    t1 = jnp.maximum(jnp.dot(z, w1_ref[...],
                             preferred_element_type=jnp.float32) + b1_ref[...],
                     0.0)
    u = jnp.maximum(jnp.dot(t1, w2_ref[...],
                            preferred_element_type=jnp.float32) + b2_ref[...],
                    0.0)
    rows = i * BM + lax.broadcasted_iota(jnp.int32, (BM, 1), 0)
    u = jnp.where(rows < N_NODES, u, 0.0)
    u_ref[...] = u

    acc = acc_ref[...]
    for j in range(16):
        acc = acc + u[8 * j:8 * j + 8, :]
    acc_ref[...] = acc

    @pl.when(i == NB - 1)
    def _():
        sums_ref[...] = acc_ref[...]


def _mlp(h, agg, W1, b1, W2, b2):
    return pl.pallas_call(
        _mlp_body,
        grid=(NB,),
        in_specs=[
            pl.BlockSpec((BM, D), lambda i: (i, 0)),
            pl.BlockSpec((2, BM, D), lambda i: (0, i, 0)),
            pl.BlockSpec((D, D), lambda i: (0, 0)),
            pl.BlockSpec((1, D), lambda i: (0, 0)),
            pl.BlockSpec((D, D), lambda i: (0, 0)),
            pl.BlockSpec((1, D), lambda i: (0, 0)),
        ],
        out_specs=[
            pl.BlockSpec((BM, D), lambda i: (i, 0)),
            pl.BlockSpec((8, D), lambda i: (0, 0)),
        ],
        out_shape=[
            jax.ShapeDtypeStruct((NROWS, D), jnp.float32),
            jax.ShapeDtypeStruct((8, D), jnp.float32),
        ],
        scratch_shapes=[pltpu.VMEM((8, D), jnp.float32)],
    )(h, agg, W1, b1, W2, b2)


def _var_body(u_ref, sums_ref, v0_ref, v1_ref, a0_ref, a1_ref):
    i = pl.program_id(0)

    @pl.when(i == 0)
    def _():
        a0_ref[...] = jnp.zeros_like(a0_ref)
        a1_ref[...] = jnp.zeros_like(a1_ref)

    m = _tree8(sums_ref[...]) * jnp.float32(1e-4)
    rows = i * BM + lax.broadcasted_iota(jnp.int32, (BM, 1), 0)
    dev = u_ref[...] - m
    sq = jnp.where(rows < N_NODES, dev * dev, 0.0)

    @pl.when(i < 39)
    def _():
        acc = a0_ref[...]
        for j in range(16):
            acc = acc + sq[8 * j:8 * j + 8, :]
        a0_ref[...] = acc

    @pl.when(i == 39)
    def _():
        a0_ref[...] = a0_ref[...] + sq[0:8, :]
        acc = a1_ref[...]
        for j in range(1, 16):
            acc = acc + sq[8 * j:8 * j + 8, :]
        a1_ref[...] = acc

    @pl.when(i > 39)
    def _():
        acc = a1_ref[...]
        for j in range(16):
            acc = acc + sq[8 * j:8 * j + 8, :]
        a1_ref[...] = acc

    @pl.when(i == NB - 1)
    def _():
        v0_ref[...] = a0_ref[...]
        v1_ref[...] = a1_ref[...]


def _var(u, sums):
    return pl.pallas_call(
        _var_body,
        grid=(NB,),
        in_specs=[
            pl.BlockSpec((BM, D), lambda i: (i, 0)),
            pl.BlockSpec((8, D), lambda i: (0, 0)),
        ],
        out_specs=[
            pl.BlockSpec((8, D), lambda i: (0, 0)),
            pl.BlockSpec((8, D), lambda i: (0, 0)),
        ],
        out_shape=[
            jax.ShapeDtypeStruct((8, D), jnp.float32),
            jax.ShapeDtypeStruct((8, D), jnp.float32),
        ],
        scratch_shapes=[
            pltpu.VMEM((8, D), jnp.float32),
            pltpu.VMEM((8, D), jnp.float32),
        ],
    )(u, sums)


def _norm_body(u_ref, sums_ref, v0_ref, v1_ref, g_ref, b_ref, out_ref):
    i = pl.program_id(0)
    m = _tree8(sums_ref[...]) * jnp.float32(1e-4)
    vs = _tree8(v0_ref[...]) + _tree8(v1_ref[...])
    den = jnp.sqrt(vs * jnp.float32(1e-4) + jnp.float32(1e-5))
    out = (u_ref[...] - m) / den * g_ref[...] + b_ref[...]
    rows = i * BM + lax.broadcasted_iota(jnp.int32, (BM, 1), 0)
    out_ref[...] = jnp.where(rows < N_NODES, out, 0.0)


def _norm(u, sums, v0, v1, g, b):
    return pl.pallas_call(
        _norm_body,
        grid=(NB,),
        in_specs=[
            pl.BlockSpec((BM, D), lambda i: (i, 0)),
            pl.BlockSpec((8, D), lambda i: (0, 0)),
            pl.BlockSpec((8, D), lambda i: (0, 0)),
            pl.BlockSpec((8, D), lambda i: (0, 0)),
            pl.BlockSpec((1, D), lambda i: (0, 0)),
            pl.BlockSpec((1, D), lambda i: (0, 0)),
        ],
        out_specs=pl.BlockSpec((BM, D), lambda i: (i, 0)),
        out_shape=jax.ShapeDtypeStruct((NROWS, D), jnp.float32),
    )(u, sums, v0, v1, g, b)


def _fc_body(ps_ref, batch_ref, wfc_ref, bfc_ref, out_ref, cnt_ref):
    i = pl.program_id(0)

    @pl.when(i == 0)
    def _():
        cnt_ref[...] = jnp.zeros_like(cnt_ref)

    bids = batch_ref[0]
    g_col = lax.broadcasted_iota(jnp.int32, (N_GRAPHS, 1), 0)
    mt = (g_col == bids).astype(jnp.float32)
    cnt_ref[...] += jnp.sum(mt, axis=1, keepdims=True)

    @pl.when(i == NB - 1)
    def _():
        pooled = ps_ref[...] / jnp.maximum(cnt_ref[...], 1.0)
        out = jnp.dot(pooled, wfc_ref[...],
                      preferred_element_type=jnp.float32)
        out_ref[...] = jnp.tanh(out + bfc_ref[...])


def _fc(pooled_sum, batch_r, Wfc, bfc):
    return pl.pallas_call(
        _fc_body,
        grid=(NB,),
        in_specs=[
            pl.BlockSpec((N_GRAPHS, D), lambda i: (0, 0)),
            pl.BlockSpec((1, 1, BM), lambda i: (i, 0, 0)),
            pl.BlockSpec((D, D), lambda i: (0, 0)),
            pl.BlockSpec((1, D), lambda i: (0, 0)),
        ],
        out_specs=pl.BlockSpec((N_GRAPHS, D), lambda i: (0, 0)),
        out_shape=jax.ShapeDtypeStruct((N_GRAPHS, D), jnp.float32),
        scratch_shapes=[pltpu.VMEM((N_GRAPHS, 1), jnp.float32)],
    )(pooled_sum, batch_r, Wfc, bfc)


def kernel(x, edge_index, batch, params):
    src = edge_index[0]
    dst = edge_index[1]
    perm = jnp.argsort(dst, stable=True).astype(jnp.int32)
    perm_pad = jnp.concatenate(
        [perm, jnp.zeros((E_PAD - E,), jnp.int32)])
    ssrc, sdst = _sc_prep(src, dst, perm_pad)

    zflat = jnp.zeros((NROWS * D,), jnp.float32)
    h = jnp.zeros((NROWS, D), jnp.float32).at[:N_NODES].set(x)
    batch_pad = jnp.concatenate(
        [batch, jnp.full((NROWS - N_NODES,), N_GRAPHS, jnp.int32)])
    batch_r = batch_pad.reshape(NB, 1, BM)

    for l in range(5):
        W1, b1, W2, b2 = params["convs"][l]
        g, bb = params["bns"][l]
        agg = _sc_agg(h, ssrc, sdst, zflat).reshape(2, NROWS, D)
        u, sums = _mlp(h, agg, W1, b1.reshape(1, D), W2, b2.reshape(1, D))
        v0, v1 = _var(u, sums)
        h = _norm(u, sums, v0, v1, g.reshape(1, D), bb.reshape(1, D))

    pooled_sum = _sc_pool(h, batch_pad, zflat).reshape(N_GRAPHS, D)
    Wfc, bfc = params["fc"]
    return _fc(pooled_sum, batch_r, Wfc, bfc.reshape(1, D))
